# baseline (speedup 1.0000x reference)
import functools

import jax
import jax.numpy as jnp
from jax import lax
from jax.experimental import pallas as pl
from jax.experimental.pallas import tpu as pltpu
from jax.experimental.pallas import tpu_sc as plsc

VOCAB = 1000000
BATCH = 16384
DIM = 16

_INFO = plsc.get_sparse_core_info()
_NC = _INFO.num_cores
_NS = _INFO.num_subcores
_NL = _INFO.num_lanes
_NW = _NC * _NS
_B_PER_W = BATCH // _NW
_GROUP = 128 // DIM
_NCHUNK = 4
_CH = _B_PER_W // _NCHUNK
_NBUF = 2


def _gather_body(idx_hbm, table_hbm, out_hbm, idx_v, rowidx_v,
                 rows_v, out_v, sem0, sem1):
    sems = (sem0, sem1)
    wid = lax.axis_index("s") * _NC + lax.axis_index("c")
    base = wid * _B_PER_W

    pltpu.sync_copy(idx_hbm.at[pl.ds(base, _B_PER_W)], idx_v)

    def split(g, _):
        chunk = idx_v[pl.ds(g * _NL, _NL)]
        rowidx_v[pl.ds(g * _NL, _NL)] = lax.shift_right_logical(chunk, 3)
        idx_v[pl.ds(g * _NL, _NL)] = lax.shift_left(
            lax.bitwise_and(chunk, _GROUP - 1), 4)
        return 0

    lax.fori_loop(0, _B_PER_W // _NL, split, 0, unroll=4)

    def fire(c):
        return pltpu.async_copy(
            table_hbm.at[rowidx_v.at[pl.ds(c * _CH, _CH)]],
            rows_v.at[c % _NBUF], sems[c % _NBUF])

    copies = [None] * _NCHUNK
    for c in range(_NBUF):
        copies[c] = fire(c)
    lanes = lax.iota(jnp.int32, _NL)
    for c in range(_NCHUNK):
        copies[c].wait()
        buf = jnp.full((_NL,), c % _NBUF, jnp.int32)

        def extract(g, _):
            row_l = g * _NL + lanes
            row_g = c * _CH + row_l
            sub = idx_v[pl.ds(c * _CH + g * _NL, _NL)]
            for d in range(DIM):
                vals = plsc.load_gather(rows_v, [buf, row_l, sub + d])
                plsc.store_scatter(
                    out_v, [row_g, jnp.full((_NL,), d, jnp.int32)], vals)
            return 0

        lax.fori_loop(0, _CH // _NL, extract, 0)
        if c + _NBUF < _NCHUNK:
            copies[c + _NBUF] = fire(c + _NBUF)

    pltpu.sync_copy(out_v, out_hbm.at[pl.ds(base, _B_PER_W)])


@jax.jit
def kernel(index, train_score):
    mesh = plsc.VectorSubcoreMesh(core_axis_name="c", subcore_axis_name="s")
    table = jnp.reshape(train_score, (VOCAB // _GROUP, 128))
    k = functools.partial(
        pl.kernel,
        mesh=mesh,
        out_type=jax.ShapeDtypeStruct((BATCH, DIM), jnp.float32),
        scratch_types=[
            pltpu.VMEM((_B_PER_W,), jnp.int32),
            pltpu.VMEM((_B_PER_W,), jnp.int32),
            pltpu.VMEM((_NBUF, _CH, 128), jnp.float32),
            pltpu.VMEM((_B_PER_W, DIM), jnp.float32),
            pltpu.SemaphoreType.DMA,
            pltpu.SemaphoreType.DMA,
        ],
        compiler_params=pltpu.CompilerParams(
            needs_layout_passes=False, use_tc_tiling_on_sc=True),
    )(_gather_body)
    return k(index, table)


# speedup vs baseline: 4.7668x; 4.7668x over previous
import functools

import jax
import jax.numpy as jnp
from jax import lax
from jax.experimental import pallas as pl
from jax.experimental.pallas import tpu as pltpu
from jax.experimental.pallas import tpu_sc as plsc

VOCAB = 1000000
BATCH = 16384
DIM = 16

_INFO = plsc.get_sparse_core_info()
_NC = _INFO.num_cores
_NS = _INFO.num_subcores
_NW = _NC * _NS
_B_PER_W = BATCH // _NW
_DEPTH = 16


def _gather_body(idx_hbm, table_hbm, out_hbm, idx_v, blk_v, col_v, *sems):
    wid = lax.axis_index("s") * _NC + lax.axis_index("c")
    base = wid * _B_PER_W

    pltpu.sync_copy(idx_hbm.at[pl.ds(base, _B_PER_W)], idx_v)

    dims = lax.iota(jnp.int32, DIM)

    def group(g, _):
        b0 = g * _DEPTH
        vec = idx_v[pl.ds(b0, _DEPTH)]
        blk = lax.min(
            lax.shift_left(lax.shift_right_logical(vec, 7), 7),
            jnp.full((_DEPTH,), VOCAB - 128, jnp.int32))
        copies = [
            pltpu.async_copy(
                table_hbm.at[:, pl.ds(pl.multiple_of(blk[i], 128), 128)],
                blk_v.at[i], sems[i])
            for i in range(_DEPTH)
        ]
        lane = vec - blk
        for i in range(_DEPTH):
            copies[i].wait()
            vals = plsc.load_gather(
                blk_v, [jnp.full((DIM,), i, jnp.int32), dims,
                        jnp.full((DIM,), lane[i], jnp.int32)])
            plsc.store_scatter(
                col_v, [dims, jnp.full((DIM,), b0 + i, jnp.int32)], vals)
        return 0

    lax.fori_loop(0, _B_PER_W // _DEPTH, group, 0)

    pltpu.sync_copy(col_v, out_hbm.at[:, pl.ds(base, _B_PER_W)])


@jax.jit
def kernel(index, train_score):
    mesh = plsc.VectorSubcoreMesh(core_axis_name="c", subcore_axis_name="s")
    table_t = jnp.transpose(train_score)
    k = functools.partial(
        pl.kernel,
        mesh=mesh,
        out_type=jax.ShapeDtypeStruct((DIM, BATCH), jnp.float32),
        scratch_types=[
            pltpu.VMEM((_B_PER_W,), jnp.int32),
            pltpu.VMEM((_DEPTH, DIM, 128), jnp.float32),
            pltpu.VMEM((DIM, _B_PER_W), jnp.float32),
        ] + [pltpu.SemaphoreType.DMA] * _DEPTH,
        compiler_params=pltpu.CompilerParams(
            needs_layout_passes=False, use_tc_tiling_on_sc=True),
    )(_gather_body)
    return jnp.transpose(k(index, table_t))


# speedup vs baseline: 4.7681x; 1.0003x over previous
import functools

import jax
import jax.numpy as jnp
from jax import lax
from jax.experimental import pallas as pl
from jax.experimental.pallas import tpu as pltpu
from jax.experimental.pallas import tpu_sc as plsc

VOCAB = 1000000
BATCH = 16384
DIM = 16

_INFO = plsc.get_sparse_core_info()
_NC = _INFO.num_cores
_NS = _INFO.num_subcores
_NW = _NC * _NS
_B_PER_W = BATCH // _NW
_DEPTH = 16


def _gather_body(idx_hbm, table_hbm, out_hbm, idx_v, blk_v, col_v, *sems):
    wid = lax.axis_index("s") * _NC + lax.axis_index("c")
    base = wid * _B_PER_W

    pltpu.sync_copy(idx_hbm.at[pl.ds(base, _B_PER_W)], idx_v)

    dims = lax.iota(jnp.int32, DIM)
    ngroups = _B_PER_W // _DEPTH

    def group(g, _):
        b0 = g * _DEPTH
        vec = idx_v[pl.ds(b0, _DEPTH)]
        blk = lax.shift_left(lax.shift_right_logical(vec, 7), 7)
        copies = [
            pltpu.async_copy(
                table_hbm.at[:, pl.ds(pl.multiple_of(blk[i], 128), 128)],
                blk_v.at[i], sems[i])
            for i in range(_DEPTH)
        ]
        lane = vec - blk
        for i in range(_DEPTH):
            copies[i].wait()
            vals = plsc.load_gather(
                blk_v, [jnp.full((DIM,), i, jnp.int32), dims,
                        jnp.full((DIM,), lane[i], jnp.int32)])
            plsc.store_scatter(
                col_v, [dims, jnp.full((DIM,), b0 + i, jnp.int32)], vals)
        return 0

    lax.fori_loop(0, ngroups, group, 0)

    pltpu.sync_copy(col_v, out_hbm.at[:, pl.ds(base, _B_PER_W)])


@jax.jit
def kernel(index, train_score):
    mesh = plsc.VectorSubcoreMesh(core_axis_name="c", subcore_axis_name="s")
    table_t = jnp.transpose(train_score)
    k = functools.partial(
        pl.kernel,
        mesh=mesh,
        out_type=jax.ShapeDtypeStruct((DIM, BATCH), jnp.float32),
        scratch_types=[
            pltpu.VMEM((_B_PER_W,), jnp.int32),
            pltpu.VMEM((_DEPTH, DIM, 128), jnp.float32),
            pltpu.VMEM((DIM, _B_PER_W), jnp.float32),
        ] + [pltpu.SemaphoreType.DMA] * _DEPTH,
        compiler_params=pltpu.CompilerParams(
            needs_layout_passes=False, use_tc_tiling_on_sc=True),
    )(_gather_body)
    return jnp.transpose(k(index, table_t))
